# baseline (speedup 1.0000x reference)
import jax
import jax.numpy as jnp
from jax.experimental import pallas as pl
from jax.experimental.pallas import tpu as pltpu

_N = 8192
_BI = 256
_NI = _N // _BI


def _chamfer_body(t_ref, ot_ref, out_ref, d2_ref, acc_ref):
    i = pl.program_id(0)
    t = t_ref[...]
    ot = ot_ref[...]
    t2 = jnp.sum(t * t, axis=1, keepdims=True)
    o2 = jnp.sum(ot * ot, axis=0, keepdims=True)
    dot = jax.lax.dot_general(
        t, ot, (((1,), (0,)), ((), ())), preferred_element_type=jnp.float32
    )
    d = jnp.maximum(t2 + o2 - 2.0 * dot, 0.0)
    d1 = jnp.min(d, axis=1)
    sq = jnp.sum(jnp.sqrt(d1))
    cmin = jnp.min(d, axis=0, keepdims=True)

    @pl.when(i == 0)
    def _():
        acc_ref[0, 0] = sq
        d2_ref[...] = cmin

    @pl.when(i > 0)
    def _():
        acc_ref[0, 0] = acc_ref[0, 0] + sq
        d2_ref[...] = jnp.minimum(d2_ref[...], cmin)

    @pl.when(i == _NI - 1)
    def _():
        s2 = jnp.sum(jnp.sqrt(d2_ref[...]))
        loss = (acc_ref[0, 0] / _N + s2 / _N) * 5.0
        out_ref[...] = jnp.full((1, 1), loss, jnp.float32)


def kernel(target, output):
    ot = output.T
    out = pl.pallas_call(
        _chamfer_body,
        grid=(_NI,),
        in_specs=[
            pl.BlockSpec((_BI, 3), lambda i: (i, 0)),
            pl.BlockSpec((3, _N), lambda i: (0, 0)),
        ],
        out_specs=pl.BlockSpec((1, 1), lambda i: (0, 0)),
        out_shape=jax.ShapeDtypeStruct((1, 1), jnp.float32),
        scratch_shapes=[
            pltpu.VMEM((1, _N), jnp.float32),
            pltpu.SMEM((1, 1), jnp.float32),
        ],
    )(target, ot)
    return out[0, 0]


# speedup vs baseline: 1.5161x; 1.5161x over previous
import jax
import jax.numpy as jnp
from jax.experimental import pallas as pl
from jax.experimental.pallas import tpu as pltpu

_N = 8192
_BI = 256
_NI = _N // _BI


def _chamfer_body(t_ref, ot_ref, out_ref, b_ref, d2_ref, acc_ref):
    i = pl.program_id(0)

    @pl.when(i == 0)
    def _():
        ot = ot_ref[...]
        o2 = jnp.sum(ot * ot, axis=0, keepdims=True)
        b_ref[...] = jnp.concatenate(
            [jnp.ones((3, _N), jnp.float32), -2.0 * ot, o2], axis=0
        )

    t = t_ref[...]
    a = jnp.concatenate(
        [t * t, t, jnp.ones((_BI, 1), jnp.float32)], axis=1
    )
    d = jax.lax.dot_general(
        a, b_ref[...], (((1,), (0,)), ((), ())),
        preferred_element_type=jnp.float32,
    )
    d1 = jnp.maximum(jnp.min(d, axis=1), 0.0)
    sq = jnp.sum(jnp.sqrt(d1))
    cmin = jnp.min(d, axis=0, keepdims=True)

    @pl.when(i == 0)
    def _():
        acc_ref[0, 0] = sq
        d2_ref[...] = cmin

    @pl.when(i > 0)
    def _():
        acc_ref[0, 0] = acc_ref[0, 0] + sq
        d2_ref[...] = jnp.minimum(d2_ref[...], cmin)

    @pl.when(i == _NI - 1)
    def _():
        d2 = jnp.maximum(d2_ref[...], 0.0)
        s2 = jnp.sum(jnp.sqrt(d2))
        loss = (acc_ref[0, 0] / _N + s2 / _N) * 5.0
        out_ref[...] = jnp.full((1, 1), loss, jnp.float32)


def kernel(target, output):
    ot = output.T
    out = pl.pallas_call(
        _chamfer_body,
        grid=(_NI,),
        in_specs=[
            pl.BlockSpec((_BI, 3), lambda i: (i, 0)),
            pl.BlockSpec((3, _N), lambda i: (0, 0)),
        ],
        out_specs=pl.BlockSpec((1, 1), lambda i: (0, 0)),
        out_shape=jax.ShapeDtypeStruct((1, 1), jnp.float32),
        scratch_shapes=[
            pltpu.VMEM((7, _N), jnp.float32),
            pltpu.VMEM((1, _N), jnp.float32),
            pltpu.SMEM((1, 1), jnp.float32),
        ],
    )(target, ot)
    return out[0, 0]
